# baseline (speedup 1.0000x reference)
import functools

import jax
import jax.numpy as jnp
from jax import lax
from jax.experimental import pallas as pl
from jax.experimental.pallas import tpu as pltpu
from jax.experimental.pallas import tpu_sc as plsc

B = 1024
T = 50
H = 128
STAT = 8
MULTI = 8
TGT = 4
NCAT = 3
VOCAB = 100
VPAD = 104

BB = 32
RB = BB * T
MT = RB * TGT
MK = RB * MULTI

NG = B * T * NCAT
NW = 32
PER_W = NG // NW
CH = 120
NIT = PER_W // CH


def _hilo(x):
    hi = x.astype(jnp.bfloat16)
    lo = (x - hi.astype(jnp.float32)).astype(jnp.bfloat16)
    return hi, lo


def _tc_body(meT_ref, tval_ref, xc_ref, stat_ref, sv_ref, sb_ref, mv_ref,
             mb_ref, tv_ref, tb_ref, e0_ref, e1_ref, e2_ref,
             s_out, k_out, t_out, gtab_out, idx_out, dst_out,
             tw, kw, tind, kind, cmask):
    i = pl.program_id(0)

    @pl.when(i == 0)
    def _():
        for c, e_ref in enumerate((e0_ref, e1_ref, e2_ref)):
            e = e_ref[...]
            g = 0.5 * e * (1.0 + jax.lax.erf(e * 0.7071067811865476))
            gtab_out[c * VPAD:c * VPAD + VOCAB, :] = g
        tvh, tvl = _hilo(tv_ref[...])
        tbh, tbl = _hilo(tb_ref[...])
        tw[...] = jnp.concatenate([tvh, tvl, tbh, tbl], axis=0)
        mvh, mvl = _hilo(mv_ref[NCAT:NCAT + 1, :])
        mbh, mbl = _hilo(mb_ref[NCAT:, :])
        kw[...] = jnp.concatenate([mvh, mvl, mbh, mbl], axis=0)
        lane_t = jax.lax.broadcasted_iota(jnp.int32, (2 * TGT, MT), 1) % TGT
        row_t = jax.lax.broadcasted_iota(jnp.int32, (2 * TGT, MT), 0) % TGT
        tind[...] = (lane_t == row_t).astype(jnp.bfloat16)
        lane_k = jax.lax.broadcasted_iota(jnp.int32, (2 * (MULTI - NCAT), MK),
                                          1) % MULTI
        row_k = jax.lax.broadcasted_iota(jnp.int32, (2 * (MULTI - NCAT), MK),
                                         0) % (MULTI - NCAT)
        kind[...] = (lane_k == NCAT + row_k).astype(jnp.bfloat16)
        lane_c = jax.lax.broadcasted_iota(jnp.int32, (1, MK), 1) % MULTI
        cmask[...] = (lane_c >= NCAT).astype(jnp.bfloat16)

    meT = meT_ref[0]
    crow = jax.lax.broadcasted_iota(jnp.int32, (NCAT, RB), 0)
    ilane = jax.lax.broadcasted_iota(jnp.int32, (NCAT, RB), 1)
    idx_out[0] = jnp.floor(meT).astype(jnp.int32) + crow * VPAD
    dst_out[0] = (i * RB + ilane) * MULTI + crow

    stat = stat_ref[...]
    s_out[...] = stat[:, :, None] * sv_ref[...][None] + sb_ref[...][None]

    xf = tval_ref[0].astype(jnp.bfloat16)
    tval = tind[:TGT, :] * xf
    t_lhs = jnp.concatenate([tval, tval, tind[...]], axis=0)
    t_int = jax.lax.dot_general(
        t_lhs, tw[...], (((0,), (0,)), ((), ())),
        preferred_element_type=jnp.float32)
    t_out[...] = t_int.reshape(RB, TGT, H)

    mf = xc_ref[0].astype(jnp.bfloat16)
    xc = cmask[...] * mf
    k_lhs = jnp.concatenate([xc, xc, kind[...]], axis=0)
    k_int = jax.lax.dot_general(
        k_lhs, kw[...], (((0,), (0,)), ((), ())),
        preferred_element_type=jnp.float32)
    k_out[...] = k_int.reshape(RB, MULTI, H)


def _sc_body(gtab_hbm, idx_hbm, dst_hbm, k_hbm, tok_out,
             idx_v, dst_v, rows_v, sem):
    wid = lax.axis_index("s") * 2 + lax.axis_index("c")

    def it_body(it, carry):
        base = wid * PER_W + it * CH
        pltpu.sync_copy(idx_hbm.at[pl.ds(base, CH)], idx_v)
        pltpu.sync_copy(dst_hbm.at[pl.ds(base, CH)], dst_v)
        pltpu.async_copy(gtab_hbm.at[idx_v], rows_v, sem).wait()
        pltpu.async_copy(rows_v, k_hbm.at[dst_v], sem).wait()
        return carry

    lax.fori_loop(0, NIT, it_body, 0)


_sc_gather = functools.partial(
    pl.kernel,
    out_type=jax.ShapeDtypeStruct((8, 128), jnp.float32),
    mesh=plsc.VectorSubcoreMesh(core_axis_name="c", subcore_axis_name="s"),
    scratch_types=[pltpu.VMEM((CH,), jnp.int32),
                   pltpu.VMEM((CH,), jnp.int32),
                   pltpu.VMEM((CH, H), jnp.float32),
                   pltpu.SemaphoreType.DMA],
    compiler_params=pltpu.CompilerParams(has_side_effects=True),
)(_sc_body)


@jax.jit
def kernel(target_inp, stat_exog, multi_exog, stat_vec, stat_bias, multi_vec,
           multi_bias, tgt_vec, tgt_bias, emb0, emb1, emb2):
    nsteps = B // BB
    me2 = multi_exog.reshape(B * T, MULTI)
    meT = (me2[:, :NCAT].reshape(nsteps, RB, NCAT)
           .transpose(0, 2, 1).copy())
    tval = target_inp.reshape(nsteps, 1, MT)
    xc = multi_exog.reshape(nsteps, 1, MK)

    full = lambda shape: pl.BlockSpec(shape, lambda i: (0,) * len(shape))

    s2, k2, t2, gtab, idxg, dstg = pl.pallas_call(
        _tc_body,
        grid=(nsteps,),
        in_specs=[
            pl.BlockSpec((1, NCAT, RB), lambda i: (i, 0, 0)),
            pl.BlockSpec((1, 1, MT), lambda i: (i, 0, 0)),
            pl.BlockSpec((1, 1, MK), lambda i: (i, 0, 0)),
            pl.BlockSpec((BB, STAT), lambda i: (i, 0)),
            full((STAT, H)), full((STAT, H)),
            full((MULTI, H)), full((MULTI, H)),
            full((TGT, H)), full((TGT, H)),
            full((VOCAB, H)), full((VOCAB, H)), full((VOCAB, H)),
        ],
        out_specs=[
            pl.BlockSpec((BB, STAT, H), lambda i: (i, 0, 0)),
            pl.BlockSpec((RB, MULTI, H), lambda i: (i, 0, 0)),
            pl.BlockSpec((RB, TGT, H), lambda i: (i, 0, 0)),
            full((NCAT * VPAD, H)),
            pl.BlockSpec((1, NCAT, RB), lambda i: (i, 0, 0)),
            pl.BlockSpec((1, NCAT, RB), lambda i: (i, 0, 0)),
        ],
        out_shape=[
            jax.ShapeDtypeStruct((B, STAT, H), jnp.float32),
            jax.ShapeDtypeStruct((B * T, MULTI, H), jnp.float32),
            jax.ShapeDtypeStruct((B * T, TGT, H), jnp.float32),
            jax.ShapeDtypeStruct((NCAT * VPAD, H), jnp.float32),
            jax.ShapeDtypeStruct((nsteps, NCAT, RB), jnp.int32),
            jax.ShapeDtypeStruct((nsteps, NCAT, RB), jnp.int32),
        ],
        scratch_shapes=[pltpu.VMEM((4 * TGT, H), jnp.bfloat16),
                        pltpu.VMEM((2 + 2 * (MULTI - NCAT), H), jnp.bfloat16),
                        pltpu.VMEM((2 * TGT, MT), jnp.bfloat16),
                        pltpu.VMEM((2 * (MULTI - NCAT), MK), jnp.bfloat16),
                        pltpu.VMEM((1, MK), jnp.bfloat16)],
    )(meT, tval, xc, stat_exog, stat_vec, stat_bias, multi_vec, multi_bias,
      tgt_vec, tgt_bias, emb0, emb1, emb2)

    tok = _sc_gather(gtab, idxg.reshape(NG), dstg.reshape(NG),
                     k2.reshape(B * T * MULTI, H))
    k2b, _ = jax.lax.optimization_barrier((k2, tok))

    return (s2, k2b.reshape(B, T, MULTI, H), t2.reshape(B, T, TGT, H))
